# Initial kernel scaffold; baseline (speedup 1.0000x reference)
import jax
import jax.numpy as jnp
from jax.experimental import pallas as pl


def kernel(emb, W1, W2, neigh, nodes):
    raise NotImplementedError("write your pallas kernel here")


# speedup vs baseline: 3.3993x; 3.3993x over previous
import functools

import jax
import jax.numpy as jnp
from jax import lax
from jax.experimental import pallas as pl
from jax.experimental.pallas import tpu as pltpu
from jax.experimental.pallas import tpu_sc as plsc

_R, _N, _S = 4, 50000, 10
_B, _F, _H = 512, 256, 256
_M = _B * _S
_RM = _R * _M
_NC, _NS = 2, 16
_NW = _NC * _NS
_PER_W = _RM // _NW
_CH = 16
_NCH = _PER_W // _CH
_SPC = 2
_IPS = _CH * _S // _SPC


def _sc_gather_sum(emb, idx1):
    mesh = plsc.VectorSubcoreMesh(core_axis_name="c", subcore_axis_name="s")

    @functools.partial(
        pl.kernel,
        out_type=jax.ShapeDtypeStruct((_RM, _F), jnp.float32),
        mesh=mesh,
        scratch_types=[
            pltpu.VMEM((_SPC * _IPS,), jnp.int32),
            pltpu.VMEM((_CH * _S, _F), jnp.float32),
            pltpu.VMEM((_CH, _F), jnp.float32),
            pltpu.SemaphoreType.DMA,
        ],
    )
    def k(emb_hbm, idx_hbm, out_hbm, idx_v, rows_v, out_v, sem):
        wid = lax.axis_index("s") * _NC + lax.axis_index("c")
        base = wid * _PER_W

        def chunk(g, carry):
            row0 = base + g * _CH
            pltpu.sync_copy(idx_hbm.at[pl.ds(row0 * _S, _SPC * _IPS)], idx_v)
            cps = [
                pltpu.async_copy(
                    emb_hbm.at[idx_v.at[pl.ds(j * _IPS, _IPS)]],
                    rows_v.at[pl.ds(j * _IPS, _IPS)],
                    sem,
                )
                for j in range(_SPC)
            ]
            for cp in cps:
                cp.wait()

            def row(i, c2):
                for f in range(_F // 16):
                    sl = pl.ds(f * 16, 16)
                    acc = rows_v[i * _S, sl]
                    for s2 in range(1, _S):
                        acc = acc + rows_v[i * _S + s2, sl]
                    out_v[i, sl] = acc
                return c2

            lax.fori_loop(0, _CH, row, 0)
            pltpu.sync_copy(out_v, out_hbm.at[pl.ds(row0, _CH)])
            return carry

        lax.fori_loop(0, _NCH, chunk, 0)

    return k(emb, idx1)


def _tc_transform(agg, w1, w2):
    def body(a_ref, w1_ref, w2_ref, o_ref):
        r = pl.program_id(0)
        e = jnp.maximum(
            jnp.dot(a_ref[0], w1_ref[0], preferred_element_type=jnp.float32), 0.0
        )
        x = e.reshape(_B, _S, _H).sum(axis=1)
        h = jnp.maximum(
            jnp.dot(x, w2_ref[0], preferred_element_type=jnp.float32), 0.0
        )

        @pl.when(r == 0)
        def _init():
            o_ref[...] = h

        @pl.when(r != 0)
        def _acc():
            o_ref[...] += h

    return pl.pallas_call(
        body,
        grid=(_R,),
        in_specs=[
            pl.BlockSpec((1, _M, _F), lambda r: (r, 0, 0)),
            pl.BlockSpec((1, _F, _H), lambda r: (r, 0, 0)),
            pl.BlockSpec((1, _H, _H), lambda r: (r, 0, 0)),
        ],
        out_specs=pl.BlockSpec((_B, _H), lambda r: (0, 0)),
        out_shape=jax.ShapeDtypeStruct((_B, _H), jnp.float32),
    )(agg, w1, w2)


def kernel(emb, W1, W2, neigh, nodes):
    nb2 = neigh[:, nodes, :]
    flat = nb2.reshape(_R, _B * _S)
    h1 = jnp.take_along_axis(neigh, flat[:, :, None], axis=1)
    idx1 = h1.reshape(_RM * _S)
    agg = _sc_gather_sum(emb, idx1)
    inv_s = jnp.float32(1.0 / _S)
    return _tc_transform(agg.reshape(_R, _M, _F), W1 * inv_s, W2 * inv_s)


# speedup vs baseline: 4.2765x; 1.2580x over previous
import functools

import jax
import jax.numpy as jnp
from jax import lax
from jax.experimental import pallas as pl
from jax.experimental.pallas import tpu as pltpu
from jax.experimental.pallas import tpu_sc as plsc

_R, _N, _S = 4, 50000, 10
_B, _F, _H = 512, 256, 256
_M = _B * _S
_RM = _R * _M
_NC, _NS = 2, 16
_NW = _NC * _NS
_PER_W = _RM // _NW
_CH = 16
_NCH = _PER_W // _CH
_SPC = 2
_IPS = _CH * _S // _SPC


def _sc_gather_sum(emb, idx1):
    mesh = plsc.VectorSubcoreMesh(core_axis_name="c", subcore_axis_name="s")

    @functools.partial(
        pl.kernel,
        out_type=jax.ShapeDtypeStruct((_RM, _F), jnp.float32),
        mesh=mesh,
        scratch_types=[
            pltpu.VMEM((_SPC * _IPS,), jnp.int32),
            pltpu.VMEM((_SPC * _IPS,), jnp.int32),
            pltpu.VMEM((2, _CH * _S, _F), jnp.float32),
            pltpu.VMEM((2, _CH, _F), jnp.float32),
            pltpu.SemaphoreType.DMA,
            pltpu.SemaphoreType.DMA,
            pltpu.SemaphoreType.DMA,
            pltpu.SemaphoreType.DMA,
        ],
    )
    def k(emb_hbm, idx_hbm, out_hbm, idx_v0, idx_v1, rows_v, out_v, g0, g1, o0, o1):
        wid = lax.axis_index("s") * _NC + lax.axis_index("c")
        base = wid * _PER_W
        gsem = (g0, g1)
        osem = (o0, o1)
        idx_vs = (idx_v0, idx_v1)

        def gather_cps(g, slot):
            return [
                pltpu.make_async_copy(
                    emb_hbm.at[idx_vs[slot].at[pl.ds(j * _IPS, _IPS)]],
                    rows_v.at[slot, pl.ds(j * _IPS, _IPS)],
                    gsem[slot],
                )
                for j in range(_SPC)
            ]

        def out_cp(g, slot):
            row0 = base + g * _CH
            return pltpu.make_async_copy(
                out_v.at[slot], out_hbm.at[pl.ds(row0, _CH)], osem[slot]
            )

        def issue(g, slot):
            row0 = base + g * _CH
            pltpu.sync_copy(
                idx_hbm.at[pl.ds(row0 * _S, _SPC * _IPS)], idx_vs[slot]
            )
            for cp in gather_cps(g, slot):
                cp.start()

        def compute(g, slot):
            @pl.when(g >= 2)
            def _drain():
                out_cp(g - 2, slot).wait()

            def row(i, c2):
                for f in range(_F // 16):
                    sl = pl.ds(f * 16, 16)
                    acc = rows_v[slot, i * _S, sl]
                    for s2 in range(1, _S):
                        acc = acc + rows_v[slot, i * _S + s2, sl]
                    out_v[slot, i, sl] = acc
                return c2

            lax.fori_loop(0, _CH, row, 0)
            out_cp(g, slot).start()

        issue(0, 0)

        def body2(h, carry):
            g = 2 * h
            for slot in range(2):
                gg = g + slot
                for cp in gather_cps(gg, slot):
                    cp.wait()

                @pl.when(gg + 1 < _NCH)
                def _next():
                    issue(gg + 1, 1 - slot)

                compute(gg, slot)
            return carry

        lax.fori_loop(0, _NCH // 2, body2, 0)
        out_cp(_NCH - 2, 0).wait()
        out_cp(_NCH - 1, 1).wait()

    return k(emb, idx1)


def _tc_transform(agg, w1, w2):
    def body(a_ref, w1_ref, w2_ref, o_ref):
        r = pl.program_id(0)
        e = jnp.maximum(
            jnp.dot(a_ref[0], w1_ref[0], preferred_element_type=jnp.float32), 0.0
        )
        x = e.reshape(_B, _S, _H).sum(axis=1)
        h = jnp.maximum(
            jnp.dot(x, w2_ref[0], preferred_element_type=jnp.float32), 0.0
        )

        @pl.when(r == 0)
        def _init():
            o_ref[...] = h

        @pl.when(r != 0)
        def _acc():
            o_ref[...] += h

    return pl.pallas_call(
        body,
        grid=(_R,),
        in_specs=[
            pl.BlockSpec((1, _M, _F), lambda r: (r, 0, 0)),
            pl.BlockSpec((1, _F, _H), lambda r: (r, 0, 0)),
            pl.BlockSpec((1, _H, _H), lambda r: (r, 0, 0)),
        ],
        out_specs=pl.BlockSpec((_B, _H), lambda r: (0, 0)),
        out_shape=jax.ShapeDtypeStruct((_B, _H), jnp.float32),
    )(agg, w1, w2)


def kernel(emb, W1, W2, neigh, nodes):
    nb2 = neigh[:, nodes, :]
    flat = nb2.reshape(_R, _B * _S)
    h1 = jnp.take_along_axis(neigh, flat[:, :, None], axis=1)
    idx1 = h1.reshape(_RM * _S)
    agg = _sc_gather_sum(emb, idx1)
    inv_s = jnp.float32(1.0 / _S)
    return _tc_transform(agg.reshape(_R, _M, _F), W1 * inv_s, W2 * inv_s)
